# baseline (speedup 1.0000x reference)
import dataclasses
import functools

import jax
import jax.numpy as jnp
from jax import lax
from jax.experimental import pallas as pl
from jax.experimental.pallas import tpu as pltpu
from jax.experimental.pallas import tpu_sc as plsc

_ALPHA = 0.2
_L = 16
_EB = 80
_NC = 2
_NS = 16


def kernel(x, edge_index, W, a):
    N, D = x.shape
    F = W.shape[1]
    E = edge_index.shape[1]
    edge_index = edge_index.astype(jnp.int32)
    aT = jnp.transpose(a.reshape(2, F))
    IDXBITS = 14
    assert N <= (1 << IDXBITS)

    RB = 1000
    assert N % RB == 0

    def prep_body(x_ref, w_ref, at_ref, h_ref, sp_ref):
        h = jnp.dot(x_ref[...], w_ref[...], preferred_element_type=jnp.float32)
        s = jnp.dot(h, at_ref[...], preferred_element_type=jnp.float32)
        u = jax.lax.bitcast_convert_type(s, jnp.uint32) + jnp.uint32(0x8000)
        sp = (u[:, 0:1] & jnp.uint32(0xFFFF0000)) | (u[:, 1:2] >> 16)
        sp_ref[...] = jax.lax.bitcast_convert_type(sp, jnp.int32)
        h_ref[...] = h

    h, spack = pl.pallas_call(
        prep_body,
        grid=(N // RB,),
        in_specs=[
            pl.BlockSpec((RB, D), lambda i: (i, 0)),
            pl.BlockSpec((D, F), lambda i: (0, 0)),
            pl.BlockSpec((F, 2), lambda i: (0, 0)),
        ],
        out_specs=[
            pl.BlockSpec((RB, F), lambda i: (i, 0)),
            pl.BlockSpec((RB, 1), lambda i: (i, 0)),
        ],
        out_shape=[
            jax.ShapeDtypeStruct((N, F), jnp.float32),
            jax.ShapeDtypeStruct((N, 1), jnp.int32),
        ],
    )(x, W, aT)
    spk = spack.reshape(N)

    src = edge_index[0]
    dst = edge_index[1]

    NBLK = E // _EB
    assert NBLK * _EB == E
    BPC = NBLK // _NC
    BPS = BPC // _NS
    assert BPS * _NS == BPC
    NPAD = ((N + _NS * 8 - 1) // (_NS * 8)) * (_NS * 8)
    RSROWS = ((N + F - 1) // F + 7) // 8 * 8
    NTOT = ((NPAD + RSROWS + _NS * 8 - 1) // (_NS * 8)) * (_NS * 8)
    RPS = NTOT // _NS

    mesh = plsc.VectorSubcoreMesh(
        core_axis_name="c", subcore_axis_name="s",
        num_cores=_NC, num_subcores=_NS)
    sc_params = pltpu.CompilerParams()
    if "needs_layout_passes" in pltpu.CompilerParams.__dataclass_fields__:
        sc_params = dataclasses.replace(sc_params, needs_layout_passes=False)

    @functools.partial(
        pl.kernel,
        compiler_params=sc_params,
        out_type=jax.ShapeDtypeStruct((_NC * NTOT, F), jnp.float32),
        mesh=mesh,
        scratch_types=[
            pltpu.VMEM_SHARED((NTOT, F), jnp.float32),
            pltpu.VMEM((N,), jnp.int32),
            pltpu.VMEM((RSROWS, F), jnp.float32),
            pltpu.VMEM((4, _EB), jnp.int32),
            pltpu.VMEM((4, _EB), jnp.int32),
            pltpu.VMEM((_EB,), jnp.float32),
            pltpu.VMEM((RSROWS,), jnp.int32),
            pltpu.VMEM((_EB, F), jnp.float32),
            pltpu.VMEM((_EB, F), jnp.float32),
            pltpu.SemaphoreType.DMA,
            pltpu.SemaphoreType.DMA,
            pltpu.SemaphoreType.DMA,
            pltpu.SemaphoreType.DMA,
            pltpu.SemaphoreType.DMA,
            pltpu.SemaphoreType.DMA,
            pltpu.SemaphoreType.DMA,
            pltpu.SemaphoreType.DMA,
        ],
    )
    def edge_kernel(h_hbm, sp_hbm, src_hbm, dst_hbm, out_hbm,
                    acc, spv, rsl, sidx, didx, wv, rsiv, rows0, rows1,
                    psem0, psem1, psem2, psem3, gsem0, gsem1, ssem0, ssem1):
        cid = lax.axis_index("c")
        sid = lax.axis_index("s")
        row0 = sid * RPS
        blk0 = (cid * _NS + sid) * BPS
        psems = (psem0, psem1, psem2, psem3)

        def fire_idx(k, slot):
            b = (blk0 + k) * _EB
            pltpu.async_copy(src_hbm.at[pl.ds(b, _EB)], sidx.at[slot],
                             psems[slot])
            pltpu.async_copy(dst_hbm.at[pl.ds(b, _EB)], didx.at[slot],
                             psems[slot])

        def wait_idx(slot):
            pltpu.make_async_copy(
                src_hbm.at[pl.ds(0, _EB)], sidx.at[slot], psems[slot]).wait()
            pltpu.make_async_copy(
                src_hbm.at[pl.ds(0, _EB)], didx.at[slot], psems[slot]).wait()

        def fire_gather(slot, rbuf, gsem):
            pltpu.async_copy(h_hbm.at[didx.at[slot]], rbuf, gsem)

        def wait_gather(rbuf, gsem):
            pltpu.make_async_copy(h_hbm.at[pl.ds(0, _EB)], rbuf, gsem).wait()

        fire_idx(0, 0)
        fire_idx(1, 1)
        fire_idx(2, 2)

        zero16 = jnp.zeros((_L,), jnp.float32)
        iota16 = lax.iota(jnp.int32, _L)
        lane0 = iota16 == 0

        @pl.loop(0, RSROWS)
        def _(r):
            for c in range(F // _L):
                rsl[r, pl.ds(c * _L, _L)] = zero16

        for g in range(RSROWS // _L):
            rsiv[pl.ds(g * _L, _L)] = iota16 + (NPAD + g * _L)

        @pl.loop(0, RPS, step=RSROWS)
        def _(r):
            pltpu.sync_copy(rsl, acc.at[pl.ds(row0 + r, RSROWS)])

        pltpu.sync_copy(sp_hbm, spv)

        wait_idx(0)
        fire_gather(0, rows0, gsem0)
        plsc.subcore_barrier()

        rows_slots = (rows0, rows1)
        gsems = (gsem0, gsem1)
        ssems = (ssem0, ssem1)

        def wait_scatter(slot):
            pltpu.make_async_copy(
                rows_slots[slot], acc.at[pl.ds(0, _EB)], ssems[slot]).wait()

        lane_masks = tuple(iota16 == l for l in range(_L))

        def compute_block(sv, dv, rbuf):
            @plsc.parallel_loop(0, _EB, _L, unroll=5)
            def _(g):
                src16 = sv[pl.ds(g, _L)]
                g1 = plsc.load_gather(spv, [src16])
                g2 = plsc.load_gather(spv, [dv[pl.ds(g, _L)]])
                s1r = plsc.bitcast(g1 & jnp.int32(-65536), jnp.float32)
                s2r = plsc.bitcast(lax.shift_left(g2, 16), jnp.float32)
                t = s1r + s2r
                w16 = jnp.exp(-jnp.maximum(t, _ALPHA * t))
                wv[pl.ds(g, _L)] = w16
                hi16 = lax.shift_right_logical(src16, 7)
                lo16 = src16 & (F - 1)
                for l in range(_L):
                    plsc.addupdate_scatter(rsl, [hi16, lo16], w16,
                                           mask=lane_masks[l])

            @plsc.parallel_loop(0, _EB, 1, unroll=16)
            def _(i):
                isp = jnp.full((_L,), i, jnp.int32)
                wspl = plsc.load_gather(wv, [isp])
                for j in range(F // _L):
                    sl = pl.ds(j * _L, _L)
                    rbuf[i, sl] = rbuf[i, sl] * wspl

        @pl.loop(0, BPS + 1, step=4)
        def _(k0):
            for s in range(4):
                k = k0 + s
                ri = (s + 1) & 1
                ii = (s + 1) & 3

                @pl.when(k + 1 < BPS)
                def _(k=k, ri=ri, ii=ii):
                    @pl.when(k >= 1)
                    def _():
                        wait_scatter(ri)

                    wait_idx(ii)
                    fire_gather(ii, rows_slots[ri], gsems[ri])

                    @pl.when(k + 3 < BPS)
                    def _(k=k, s=s):
                        fire_idx(k + 3, (s + 3) & 3)

                @pl.when(k < BPS)
                def _(k=k, s=s):
                    wait_gather(rows_slots[s & 1], gsems[s & 1])
                    compute_block(sidx.at[s & 3], didx.at[s & 3],
                                  rows_slots[s & 1])
                    pltpu.async_copy(rows_slots[s & 1], acc.at[sidx.at[s & 3]],
                                     ssems[s & 1], add=True)

        wait_scatter(1)
        wait_scatter(0)
        pltpu.sync_copy(rsl, acc.at[rsiv], add=True)
        plsc.subcore_barrier()
        pltpu.sync_copy(acc.at[pl.ds(row0, RPS)],
                        out_hbm.at[pl.ds(cid * NTOT + row0, RPS)])

    parts = edge_kernel(h, spk, src, dst)
    parts = parts.reshape(_NC, NTOT, F)

    rs_cols = jnp.transpose(
        parts[:, NPAD:NPAD + RSROWS, :].reshape(_NC, RSROWS * F)[:, :N])

    def fin_body(p_ref, rs_ref, o_ref):
        p = p_ref[0, :, :] + p_ref[1, :, :]
        rs = rs_ref[:, 0:1] + rs_ref[:, 1:2]
        z = p / (rs + 1e-16)
        o_ref[...] = jnp.where(z > 0, z, jnp.exp(z) - 1.0)

    out = pl.pallas_call(
        fin_body,
        grid=(N // RB,),
        in_specs=[
            pl.BlockSpec((_NC, RB, F), lambda i: (0, i, 0)),
            pl.BlockSpec((RB, 2), lambda i: (i, 0)),
        ],
        out_specs=pl.BlockSpec((RB, F), lambda i: (i, 0)),
        out_shape=jax.ShapeDtypeStruct((N, F), jnp.float32),
    )(parts, rs_cols)
    return out


# speedup vs baseline: 1.1691x; 1.1691x over previous
import dataclasses
import functools

import jax
import jax.numpy as jnp
from jax import lax
from jax.experimental import pallas as pl
from jax.experimental.pallas import tpu as pltpu
from jax.experimental.pallas import tpu_sc as plsc

_ALPHA = 0.2
_L = 16
_EB = 80
_NC = 2
_NS = 16


def kernel(x, edge_index, W, a):
    N, D = x.shape
    F = W.shape[1]
    E = edge_index.shape[1]
    edge_index = edge_index.astype(jnp.int32)
    aT = jnp.transpose(a.reshape(2, F))
    IDXBITS = 14
    assert N <= (1 << IDXBITS)

    RB = 1000
    assert N % RB == 0

    def prep_body(x_ref, w_ref, at_ref, h_ref, sp_ref):
        h = jnp.dot(x_ref[...], w_ref[...], preferred_element_type=jnp.float32)
        s = jnp.dot(h, at_ref[...], preferred_element_type=jnp.float32)
        u = jax.lax.bitcast_convert_type(s, jnp.uint32) + jnp.uint32(0x8000)
        sp = (u[:, 0:1] & jnp.uint32(0xFFFF0000)) | (u[:, 1:2] >> 16)
        sp_ref[...] = jax.lax.bitcast_convert_type(sp, jnp.int32)
        h_ref[...] = h

    h, spack = pl.pallas_call(
        prep_body,
        grid=(N // RB,),
        in_specs=[
            pl.BlockSpec((RB, D), lambda i: (i, 0)),
            pl.BlockSpec((D, F), lambda i: (0, 0)),
            pl.BlockSpec((F, 2), lambda i: (0, 0)),
        ],
        out_specs=[
            pl.BlockSpec((RB, F), lambda i: (i, 0)),
            pl.BlockSpec((RB, 1), lambda i: (i, 0)),
        ],
        out_shape=[
            jax.ShapeDtypeStruct((N, F), jnp.float32),
            jax.ShapeDtypeStruct((N, 1), jnp.int32),
        ],
    )(x, W, aT)
    spk = spack.reshape(N)

    src = edge_index[0]
    dst = edge_index[1]

    NBLK = E // _EB
    assert NBLK * _EB == E
    BPC = NBLK // _NC
    BPS = BPC // _NS
    assert BPS * _NS == BPC
    NPAD = ((N + _NS * 8 - 1) // (_NS * 8)) * (_NS * 8)
    RSROWS = ((N + F - 1) // F + 7) // 8 * 8
    NTOT = ((NPAD + RSROWS + _NS * 8 - 1) // (_NS * 8)) * (_NS * 8)
    RPS = NTOT // _NS

    mesh = plsc.VectorSubcoreMesh(
        core_axis_name="c", subcore_axis_name="s",
        num_cores=_NC, num_subcores=_NS)
    sc_params = pltpu.CompilerParams()
    if "needs_layout_passes" in pltpu.CompilerParams.__dataclass_fields__:
        sc_params = dataclasses.replace(sc_params, needs_layout_passes=False)

    @functools.partial(
        pl.kernel,
        compiler_params=sc_params,
        out_type=jax.ShapeDtypeStruct((_NC * NTOT, F), jnp.float32),
        mesh=mesh,
        scratch_types=[
            pltpu.VMEM_SHARED((NTOT, F), jnp.float32),
            pltpu.VMEM((N,), jnp.int32),
            pltpu.VMEM((RSROWS, F), jnp.float32),
            pltpu.VMEM((4, _EB), jnp.int32),
            pltpu.VMEM((4, _EB), jnp.int32),
            pltpu.VMEM((_EB,), jnp.float32),
            pltpu.VMEM((RSROWS,), jnp.int32),
            pltpu.VMEM((_EB, F), jnp.float32),
            pltpu.VMEM((_EB, F), jnp.float32),
            pltpu.SemaphoreType.DMA,
            pltpu.SemaphoreType.DMA,
            pltpu.SemaphoreType.DMA,
            pltpu.SemaphoreType.DMA,
            pltpu.SemaphoreType.DMA,
            pltpu.SemaphoreType.DMA,
            pltpu.SemaphoreType.DMA,
            pltpu.SemaphoreType.DMA,
        ],
    )
    def edge_kernel(h_hbm, sp_hbm, src_hbm, dst_hbm, out_hbm,
                    acc, spv, rsl, sidx, didx, wv, rsiv, rows0, rows1,
                    psem0, psem1, psem2, psem3, gsem0, gsem1, ssem0, ssem1):
        cid = lax.axis_index("c")
        sid = lax.axis_index("s")
        row0 = sid * RPS
        blk0 = (cid * _NS + sid) * BPS
        psems = (psem0, psem1, psem2, psem3)

        def fire_idx(k, slot):
            b = (blk0 + k) * _EB
            pltpu.async_copy(src_hbm.at[pl.ds(b, _EB)], sidx.at[slot],
                             psems[slot])
            pltpu.async_copy(dst_hbm.at[pl.ds(b, _EB)], didx.at[slot],
                             psems[slot])

        def wait_idx(slot):
            pltpu.make_async_copy(
                src_hbm.at[pl.ds(0, _EB)], sidx.at[slot], psems[slot]).wait()
            pltpu.make_async_copy(
                src_hbm.at[pl.ds(0, _EB)], didx.at[slot], psems[slot]).wait()

        def fire_gather(slot, rbuf, gsem):
            pltpu.async_copy(h_hbm.at[didx.at[slot]], rbuf, gsem)

        def wait_gather(rbuf, gsem):
            pltpu.make_async_copy(h_hbm.at[pl.ds(0, _EB)], rbuf, gsem).wait()

        fire_idx(0, 0)
        fire_idx(1, 1)
        fire_idx(2, 2)

        zero16 = jnp.zeros((_L,), jnp.float32)
        iota16 = lax.iota(jnp.int32, _L)
        lane0 = iota16 == 0

        @pl.loop(0, RSROWS)
        def _(r):
            for c in range(F // _L):
                rsl[r, pl.ds(c * _L, _L)] = zero16

        for g in range(RSROWS // _L):
            rsiv[pl.ds(g * _L, _L)] = iota16 + (NPAD + g * _L)

        @pl.loop(0, RPS, step=RSROWS)
        def _(r):
            pltpu.sync_copy(rsl, acc.at[pl.ds(row0 + r, RSROWS)])

        pltpu.sync_copy(sp_hbm, spv)

        wait_idx(0)
        fire_gather(0, rows0, gsem0)
        plsc.subcore_barrier()

        rows_slots = (rows0, rows1)
        gsems = (gsem0, gsem1)
        ssems = (ssem0, ssem1)

        def wait_scatter(slot):
            pltpu.make_async_copy(
                rows_slots[slot], acc.at[pl.ds(0, _EB)], ssems[slot]).wait()

        lane_masks = tuple(iota16 == l for l in range(_L))

        def compute_block(sv, dv, rbuf):
            @plsc.parallel_loop(0, _EB, _L, unroll=5)
            def _(g):
                src16 = sv[pl.ds(g, _L)]
                g1 = plsc.load_gather(spv, [src16])
                g2 = plsc.load_gather(spv, [dv[pl.ds(g, _L)]])
                s1r = plsc.bitcast(g1 & jnp.int32(-65536), jnp.float32)
                s2r = plsc.bitcast(lax.shift_left(g2, 16), jnp.float32)
                t = s1r + s2r
                w16 = jnp.exp(-jnp.maximum(t, _ALPHA * t))
                wv[pl.ds(g, _L)] = w16
                hi16 = lax.shift_right_logical(src16, 7)
                lo16 = src16 & (F - 1)
                for l in range(_L):
                    plsc.addupdate_scatter(rsl, [hi16, lo16], w16,
                                           mask=lane_masks[l])

            @plsc.parallel_loop(0, _EB, 1, unroll=8)
            def _(i):
                isp = jnp.full((_L,), i, jnp.int32)
                wspl = plsc.load_gather(wv, [isp])
                for j in range(F // _L):
                    sl = pl.ds(j * _L, _L)
                    rbuf[i, sl] = rbuf[i, sl] * wspl

        @pl.loop(0, BPS + 1, step=4)
        def _(k0):
            for s in range(4):
                k = k0 + s
                ri = (s + 1) & 1
                ii = (s + 1) & 3

                @pl.when(k + 1 < BPS)
                def _(k=k, ri=ri, ii=ii):
                    @pl.when(k >= 1)
                    def _():
                        wait_scatter(ri)

                    wait_idx(ii)
                    fire_gather(ii, rows_slots[ri], gsems[ri])

                    @pl.when(k + 3 < BPS)
                    def _(k=k, s=s):
                        fire_idx(k + 3, (s + 3) & 3)

                @pl.when(k < BPS)
                def _(k=k, s=s):
                    wait_gather(rows_slots[s & 1], gsems[s & 1])
                    compute_block(sidx.at[s & 3], didx.at[s & 3],
                                  rows_slots[s & 1])
                    pltpu.async_copy(rows_slots[s & 1], acc.at[sidx.at[s & 3]],
                                     ssems[s & 1], add=True)

        wait_scatter(1)
        wait_scatter(0)
        pltpu.sync_copy(rsl, acc.at[rsiv], add=True)
        plsc.subcore_barrier()
        pltpu.sync_copy(acc.at[pl.ds(row0, RPS)],
                        out_hbm.at[pl.ds(cid * NTOT + row0, RPS)])

    parts = edge_kernel(h, spk, src, dst)
    parts = parts.reshape(_NC, NTOT, F)

    rs_cols = jnp.transpose(
        parts[:, NPAD:NPAD + RSROWS, :].reshape(_NC, RSROWS * F)[:, :N])

    def fin_body(p_ref, rs_ref, o_ref):
        p = p_ref[0, :, :] + p_ref[1, :, :]
        rs = rs_ref[:, 0:1] + rs_ref[:, 1:2]
        z = p / (rs + 1e-16)
        o_ref[...] = jnp.where(z > 0, z, jnp.exp(z) - 1.0)

    out = pl.pallas_call(
        fin_body,
        grid=(N // RB,),
        in_specs=[
            pl.BlockSpec((_NC, RB, F), lambda i: (0, i, 0)),
            pl.BlockSpec((RB, 2), lambda i: (i, 0)),
        ],
        out_specs=pl.BlockSpec((RB, F), lambda i: (i, 0)),
        out_shape=jax.ShapeDtypeStruct((N, F), jnp.float32),
    )(parts, rs_cols)
    return out
